# baseline (speedup 1.0000x reference)
import functools

import jax
import jax.numpy as jnp
from jax import lax
from jax.experimental import pallas as pl
from jax.experimental.pallas import tpu as pltpu
from jax.experimental.pallas import tpu_sc as plsc

T = 2048
H = 1024
NH = 16
HD = 64
E = 64
F = 512
FS = 1024
BM = 256
CAP = 20480
NT = CAP // BM
EPS = 1e-6
ROPE_BASE = 5000000.0

_f32 = jnp.float32
_i32 = jnp.int32


def _silu(x):
    return x * (1.0 / (1.0 + jnp.exp(-x)))


def _attn_body(x_ref, cos_ref, sin_ref, ln1_ref,
               wq_ref, wk_ref, wv_ref, out_ref, xn_scr):
    h = pl.program_id(0)

    @pl.when(h == 0)
    def _():
        x = x_ref[...]
        var = jnp.mean(x * x, axis=1, keepdims=True)
        xn_scr[...] = (x * lax.rsqrt(var + EPS)
                       * ln1_ref[...]).astype(jnp.bfloat16)

    xn = xn_scr[...]
    q = jnp.dot(xn, wq_ref[0].astype(jnp.bfloat16),
                preferred_element_type=_f32)
    k = jnp.dot(xn, wk_ref[0].astype(jnp.bfloat16),
                preferred_element_type=_f32)
    v = jnp.dot(xn, wv_ref[0].astype(jnp.bfloat16),
                preferred_element_type=_f32)
    cos = cos_ref[...]
    sin = sin_ref[...]

    def rope(t):
        rot = jnp.concatenate([-t[:, HD // 2:], t[:, :HD // 2]], axis=1)
        return t * cos + rot * sin

    q = rope(q).astype(jnp.bfloat16)
    k = rope(k).astype(jnp.bfloat16)
    vb = v.astype(jnp.bfloat16)

    RB = 512
    rowid = lax.broadcasted_iota(_i32, (RB, RB), 0)
    colid = lax.broadcasted_iota(_i32, (RB, RB), 1)
    for r in range(T // RB):
        cl = (r + 1) * RB
        qr = q[r * RB:cl, :]
        s = lax.dot_general(qr, k[:cl, :], (((1,), (1,)), ((), ())),
                            preferred_element_type=_f32) * (1.0 / 8.0)
        sd = jnp.where(colid > rowid, -1e30, s[:, r * RB:cl])
        if r > 0:
            s = jnp.concatenate([s[:, :r * RB], sd], axis=1)
        else:
            s = sd
        m = jnp.max(s, axis=1, keepdims=True)
        p = jnp.exp(s - m)
        denom = jnp.sum(p, axis=1, keepdims=True)
        o = jnp.dot(p.astype(jnp.bfloat16), vb[:cl, :],
                    preferred_element_type=_f32)
        out_ref[0, r * RB:cl, :] = o / denom


def _attention(x, cos, sin, ln1_w, Wq, Wk, Wv, interpret=False):
    return pl.pallas_call(
        _attn_body,
        grid=(NH,),
        in_specs=[
            pl.BlockSpec((T, H), lambda h: (0, 0)),
            pl.BlockSpec((T, HD), lambda h: (0, 0)),
            pl.BlockSpec((T, HD), lambda h: (0, 0)),
            pl.BlockSpec((1, H), lambda h: (0, 0)),
            pl.BlockSpec((1, H, HD), lambda h: (h, 0, 0)),
            pl.BlockSpec((1, H, HD), lambda h: (h, 0, 0)),
            pl.BlockSpec((1, H, HD), lambda h: (h, 0, 0)),
        ],
        out_specs=pl.BlockSpec((1, T, HD), lambda h: (h, 0, 0)),
        out_shape=jax.ShapeDtypeStruct((NH, T, HD), _f32),
        scratch_shapes=[pltpu.VMEM((T, H), jnp.bfloat16)],
        interpret=interpret,
    )(x, cos, sin, ln1_w.reshape(1, H),
      Wq.reshape(H, NH, HD).transpose(1, 0, 2),
      Wk.reshape(H, NH, HD).transpose(1, 0, 2),
      Wv.reshape(H, NH, HD).transpose(1, 0, 2))


def _route_body(attn_ref, res_ref, ln2_ref, wo_ref, wr_ref,
                h_ref, xn_ref, topw_ref, pos_ref, etile_ref):
    h = jnp.dot(attn_ref[...], wo_ref[...],
                preferred_element_type=_f32) + res_ref[...]
    h_ref[...] = h
    var = jnp.mean(h * h, axis=1, keepdims=True)
    xn = h * lax.rsqrt(var + EPS) * ln2_ref[...]
    xn_ref[...] = xn

    logits = jnp.dot(xn, wr_ref[...], preferred_element_type=_f32)
    colid = lax.broadcasted_iota(_i32, (T, E), 1)
    v0 = jnp.max(logits, axis=1, keepdims=True)
    i0 = jnp.min(jnp.where(logits == v0, colid, E), axis=1, keepdims=True)
    masked = jnp.where(colid == i0, -1e30, logits)
    v1 = jnp.max(masked, axis=1, keepdims=True)
    i1 = jnp.min(jnp.where(masked == v1, colid, E), axis=1, keepdims=True)
    w0 = 1.0 / (1.0 + jnp.exp(v1 - v0))
    topw_ref[...] = jnp.concatenate([w0, 1.0 - w0], axis=1)

    c0 = (colid == i0).astype(_f32)
    c1 = (colid == i1).astype(_f32)
    d = c0 + c1

    rid = lax.broadcasted_iota(_i32, (128, 128), 0)
    cid = lax.broadcasted_iota(_i32, (128, 128), 1)
    ltri = (cid < rid).astype(_f32)
    run = jnp.zeros((1, E), _f32)
    s_blocks = []
    for b in range(T // 128):
        db = d[b * 128:(b + 1) * 128, :]
        s_blocks.append(jnp.dot(ltri, db, preferred_element_type=_f32) + run)
        run = run + jnp.sum(db, axis=0, keepdims=True)
    s = jnp.concatenate(s_blocks, axis=0)

    counts = run
    pc = float(BM) * jnp.floor((counts + float(BM - 1)) / float(BM))
    rid_e = lax.broadcasted_iota(_i32, (E, E), 0)
    cid_e = lax.broadcasted_iota(_i32, (E, E), 1)
    ustrict = (rid_e < cid_e).astype(_f32)
    po = jnp.dot(pc, ustrict, preferred_element_type=_f32)

    pos0 = jnp.sum((po + s) * c0, axis=1, keepdims=True)
    pos1 = jnp.sum((po + s) * c1, axis=1, keepdims=True)
    pos_ref[...] = jnp.concatenate([pos0, pos1], axis=1).astype(_i32)

    po_t = jnp.reshape(po / float(BM), (E, 1))
    jf = lax.broadcasted_iota(_i32, (E, 192), 1).astype(_f32)
    cnt = jnp.sum((po_t <= jf).astype(_f32), axis=0, keepdims=True)
    etile_ref[...] = jnp.minimum(cnt - 1.0, float(E - 1)).astype(_i32)


def _route(attn, res, ln2_w, Wo, Wr, interpret=False):
    return pl.pallas_call(
        _route_body,
        out_shape=(
            jax.ShapeDtypeStruct((T, H), _f32),
            jax.ShapeDtypeStruct((T, H), _f32),
            jax.ShapeDtypeStruct((T, 2), _f32),
            jax.ShapeDtypeStruct((T, 2), _i32),
            jax.ShapeDtypeStruct((1, 192), _i32),
        ),
        interpret=interpret,
    )(attn, res, ln2_w.reshape(1, H), Wo, Wr)


def _ffn_body(et_ref, xs_ref, wg_ref, wu_ref, wd_ref, o_ref):
    xs = xs_ref[...]
    g = jnp.dot(xs, wg_ref[0], preferred_element_type=_f32)
    u = jnp.dot(xs, wu_ref[0], preferred_element_type=_f32)
    o_ref[...] = jnp.dot(_silu(g) * u, wd_ref[0], preferred_element_type=_f32)


def _grouped_ffn(etile, xs, Wg, Wu, Wd, interpret=False):
    grid_spec = pltpu.PrefetchScalarGridSpec(
        num_scalar_prefetch=1,
        grid=(NT,),
        in_specs=[
            pl.BlockSpec((BM, H), lambda j, et: (j, 0)),
            pl.BlockSpec((1, H, F), lambda j, et: (et[0, j], 0, 0)),
            pl.BlockSpec((1, H, F), lambda j, et: (et[0, j], 0, 0)),
            pl.BlockSpec((1, F, H), lambda j, et: (et[0, j], 0, 0)),
        ],
        out_specs=pl.BlockSpec((BM, H), lambda j, et: (j, 0)),
    )
    return pl.pallas_call(
        _ffn_body,
        grid_spec=grid_spec,
        out_shape=jax.ShapeDtypeStruct((CAP, H), _f32),
        interpret=interpret,
    )(etile, xs, Wg, Wu, Wd)


def _sc_dispatch(x, srcidx, pos_flat):
    mesh = plsc.VectorSubcoreMesh(core_axis_name="c", subcore_axis_name="s")
    CH = 64

    @functools.partial(
        pl.kernel,
        mesh=mesh,
        out_type=jax.ShapeDtypeStruct((CAP, H), _f32),
        scratch_types=[
            pltpu.VMEM((CH,), _i32),
            pltpu.VMEM((CH,), _i32),
            pltpu.VMEM((CH, H), _f32),
            pltpu.SemaphoreType.DMA,
            pltpu.SemaphoreType.DMA,
        ],
    )
    def k(x_ref, si_ref, pos_ref, xs_ref, idx_v, pos_v, rows_v, sem1, sem2):
        wid = lax.axis_index("s") * 2 + lax.axis_index("c")
        base = wid * 128
        for c in range(2):
            off = base + c * CH
            pltpu.sync_copy(si_ref.at[pl.ds(off, CH)], idx_v)
            pltpu.async_copy(x_ref.at[idx_v], rows_v, sem1).wait()
            pltpu.sync_copy(pos_ref.at[pl.ds(off, CH)], pos_v)
            pltpu.async_copy(rows_v, xs_ref.at[pos_v], sem2).wait()

    return k(x, srcidx, pos_flat)


def _sc_combine(o_sorted, pos0, pos1):
    mesh = plsc.VectorSubcoreMesh(core_axis_name="c", subcore_axis_name="s")
    CH = 64

    @functools.partial(
        pl.kernel,
        mesh=mesh,
        out_type=(jax.ShapeDtypeStruct((T, H), _f32),
                  jax.ShapeDtypeStruct((T, H), _f32)),
        scratch_types=[
            pltpu.VMEM((CH,), _i32),
            pltpu.VMEM((CH, H), _f32),
            pltpu.SemaphoreType.DMA,
        ],
    )
    def k(o_ref, p0_ref, p1_ref, g0_ref, g1_ref, idx_v, rows_v, sem):
        wid = lax.axis_index("s") * 2 + lax.axis_index("c")
        base = wid * CH
        pltpu.sync_copy(p0_ref.at[pl.ds(base, CH)], idx_v)
        pltpu.async_copy(o_ref.at[idx_v], rows_v, sem).wait()
        pltpu.sync_copy(rows_v, g0_ref.at[pl.ds(base, CH)])
        pltpu.sync_copy(p1_ref.at[pl.ds(base, CH)], idx_v)
        pltpu.async_copy(o_ref.at[idx_v], rows_v, sem).wait()
        pltpu.sync_copy(rows_v, g1_ref.at[pl.ds(base, CH)])

    return k(o_sorted, pos0, pos1)


def _final_body(h_ref, xn_ref, wsg_ref, wsu_ref, wsd_ref,
                g0_ref, g1_ref, tw_ref, out_ref):
    xn = xn_ref[...]
    a = jnp.dot(xn, wsg_ref[...], preferred_element_type=_f32)
    b = jnp.dot(xn, wsu_ref[...], preferred_element_type=_f32)
    sh = jnp.dot(_silu(a) * b, wsd_ref[...], preferred_element_type=_f32)
    w0 = tw_ref[:, 0:1]
    w1 = tw_ref[:, 1:2]
    out_ref[...] = h_ref[...] + sh + w0 * g0_ref[...] + w1 * g1_ref[...]


def _final(h, xn, Wsg, Wsu, Wsd, g0, g1, topw, interpret=False):
    RB = 512
    nb = T // RB
    return pl.pallas_call(
        _final_body,
        grid=(nb,),
        in_specs=[
            pl.BlockSpec((RB, H), lambda i: (i, 0)),
            pl.BlockSpec((RB, H), lambda i: (i, 0)),
            pl.BlockSpec((H, FS), lambda i: (0, 0)),
            pl.BlockSpec((H, FS), lambda i: (0, 0)),
            pl.BlockSpec((FS, H), lambda i: (0, 0)),
            pl.BlockSpec((RB, H), lambda i: (i, 0)),
            pl.BlockSpec((RB, H), lambda i: (i, 0)),
            pl.BlockSpec((RB, 2), lambda i: (i, 0)),
        ],
        out_specs=pl.BlockSpec((RB, H), lambda i: (i, 0)),
        out_shape=jax.ShapeDtypeStruct((T, H), _f32),
        interpret=interpret,
    )(h, xn, Wsg, Wsu, Wsd, g0, g1, topw)


def kernel(hidden_states, attention_mask, position_ids, ln1_w, ln2_w,
           Wq, Wk, Wv, Wo, Wr, Wg, Wu, Wd, Wsg, Wsu, Wsd):
    x = hidden_states.reshape(T, H)

    pid = position_ids.reshape(T).astype(_f32)
    inv = 1.0 / (ROPE_BASE ** (jnp.arange(0, HD, 2, dtype=_f32) / HD))
    freqs = pid[:, None] * inv
    emb = jnp.concatenate([freqs, freqs], axis=1)
    cos, sin = jnp.cos(emb), jnp.sin(emb)

    attn = _attention(x, cos, sin, ln1_w, Wq, Wk, Wv)
    attn = attn.transpose(1, 0, 2).reshape(T, H)
    h, xn, topw, pos2, etile = _route(attn, x, ln2_w, Wo, Wr)

    srcidx = (jnp.arange(2 * T, dtype=_i32) // 2)
    xs = _sc_dispatch(xn, srcidx, pos2.reshape(-1))
    o_sorted = _grouped_ffn(etile, xs, Wg, Wu, Wd)
    g0, g1 = _sc_combine(o_sorted, pos2[:, 0], pos2[:, 1])

    out = _final(h, xn, Wsg, Wsu, Wsd, g0, g1, topw)
    return out.reshape(1, T, H)


# speedup vs baseline: 1.2516x; 1.2516x over previous
import functools

import jax
import jax.numpy as jnp
from jax import lax
from jax.experimental import pallas as pl
from jax.experimental.pallas import tpu as pltpu
from jax.experimental.pallas import tpu_sc as plsc

T = 2048
H = 1024
NH = 16
HD = 64
E = 64
F = 512
FS = 1024
BM = 128
CAP = 12288
NT = CAP // BM
EPS = 1e-6
ROPE_BASE = 5000000.0

_f32 = jnp.float32
_i32 = jnp.int32


def _silu(x):
    return x * (1.0 / (1.0 + jnp.exp(-x)))


def _attn_body(x_ref, cos2_ref, sin2_ref, ln1_ref,
               wq_ref, wk_ref, wv_ref, out_ref, xn_scr):
    j = pl.program_id(0)

    @pl.when(j == 0)
    def _():
        x = x_ref[...]
        var = jnp.mean(x * x, axis=1, keepdims=True)
        xn_scr[...] = (x * lax.rsqrt(var + EPS)
                       * ln1_ref[...]).astype(jnp.bfloat16)

    xn = xn_scr[...]
    q2 = jnp.dot(xn, wq_ref[...].astype(jnp.bfloat16),
                 preferred_element_type=_f32)
    k2 = jnp.dot(xn, wk_ref[...].astype(jnp.bfloat16),
                 preferred_element_type=_f32)
    v2 = jnp.dot(xn, wv_ref[...].astype(jnp.bfloat16),
                 preferred_element_type=_f32)
    cos = cos2_ref[...]
    sin = sin2_ref[...]
    HH = HD // 2

    def rope2(t):
        rot = jnp.concatenate(
            [-t[:, HH:HD], t[:, :HH], -t[:, HD + HH:], t[:, HD:HD + HH]],
            axis=1)
        return t * cos + rot * sin

    q2 = rope2(q2).astype(jnp.bfloat16)
    k2 = rope2(k2).astype(jnp.bfloat16)
    v2 = v2.astype(jnp.bfloat16)

    RB = 512
    rowid = lax.broadcasted_iota(_i32, (RB, RB), 0)
    colid = lax.broadcasted_iota(_i32, (RB, RB), 1)
    for hh in range(2):
        q = q2[:, hh * HD:(hh + 1) * HD]
        k = k2[:, hh * HD:(hh + 1) * HD]
        vb = v2[:, hh * HD:(hh + 1) * HD]
        for r in range(T // RB):
            cl = (r + 1) * RB
            qr = q[r * RB:cl, :]
            s = lax.dot_general(qr, k[:cl, :], (((1,), (1,)), ((), ())),
                                preferred_element_type=_f32) * (1.0 / 8.0)
            sd = jnp.where(colid > rowid, -1e30, s[:, r * RB:cl])
            if r > 0:
                s = jnp.concatenate([s[:, :r * RB], sd], axis=1)
            else:
                s = sd
            m = jnp.max(s, axis=1, keepdims=True)
            p = jnp.exp(s - m)
            denom = jnp.sum(p, axis=1, keepdims=True)
            o = jnp.dot(p.astype(jnp.bfloat16), vb[:cl, :],
                        preferred_element_type=_f32)
            out_ref[r * RB:cl, hh * HD:(hh + 1) * HD] = o / denom


def _attention(x, cos, sin, ln1_w, Wq, Wk, Wv, interpret=False):
    cos2 = jnp.concatenate([cos, cos], axis=1)
    sin2 = jnp.concatenate([sin, sin], axis=1)
    return pl.pallas_call(
        _attn_body,
        grid=(NH // 2,),
        in_specs=[
            pl.BlockSpec((T, H), lambda j: (0, 0)),
            pl.BlockSpec((T, 2 * HD), lambda j: (0, 0)),
            pl.BlockSpec((T, 2 * HD), lambda j: (0, 0)),
            pl.BlockSpec((1, H), lambda j: (0, 0)),
            pl.BlockSpec((H, 2 * HD), lambda j: (0, j)),
            pl.BlockSpec((H, 2 * HD), lambda j: (0, j)),
            pl.BlockSpec((H, 2 * HD), lambda j: (0, j)),
        ],
        out_specs=pl.BlockSpec((T, 2 * HD), lambda j: (0, j)),
        out_shape=jax.ShapeDtypeStruct((T, H), _f32),
        scratch_shapes=[pltpu.VMEM((T, H), jnp.bfloat16)],
        interpret=interpret,
    )(x, cos2, sin2, ln1_w.reshape(1, H), Wq, Wk, Wv)


def _route_body(attn_ref, res_ref, ln2_ref, wo_ref, wr_ref,
                h_ref, xn_ref, topw_ref, pos_ref, etile_ref):
    h = jnp.dot(attn_ref[...], wo_ref[...],
                preferred_element_type=_f32) + res_ref[...]
    h_ref[...] = h
    var = jnp.mean(h * h, axis=1, keepdims=True)
    xn = h * lax.rsqrt(var + EPS) * ln2_ref[...]
    xn_ref[...] = xn

    logits = jnp.dot(xn, wr_ref[...], preferred_element_type=_f32)
    colid = lax.broadcasted_iota(_i32, (T, E), 1)
    v0 = jnp.max(logits, axis=1, keepdims=True)
    i0 = jnp.min(jnp.where(logits == v0, colid, E), axis=1, keepdims=True)
    masked = jnp.where(colid == i0, -1e30, logits)
    v1 = jnp.max(masked, axis=1, keepdims=True)
    i1 = jnp.min(jnp.where(masked == v1, colid, E), axis=1, keepdims=True)
    w0 = 1.0 / (1.0 + jnp.exp(v1 - v0))
    topw_ref[...] = jnp.concatenate([w0, 1.0 - w0], axis=1)

    c0 = (colid == i0).astype(_f32)
    c1 = (colid == i1).astype(_f32)
    d = c0 + c1

    rid = lax.broadcasted_iota(_i32, (128, 128), 0)
    cid = lax.broadcasted_iota(_i32, (128, 128), 1)
    ltri = (cid < rid).astype(_f32)
    run = jnp.zeros((1, E), _f32)
    s_blocks = []
    for b in range(T // 128):
        db = d[b * 128:(b + 1) * 128, :]
        s_blocks.append(jnp.dot(ltri, db, preferred_element_type=_f32) + run)
        run = run + jnp.sum(db, axis=0, keepdims=True)
    s = jnp.concatenate(s_blocks, axis=0)

    counts = run
    pc = float(BM) * jnp.floor((counts + float(BM - 1)) / float(BM))
    rid_e = lax.broadcasted_iota(_i32, (E, E), 0)
    cid_e = lax.broadcasted_iota(_i32, (E, E), 1)
    ustrict = (rid_e < cid_e).astype(_f32)
    po = jnp.dot(pc, ustrict, preferred_element_type=_f32)

    pos0 = jnp.sum((po + s) * c0, axis=1, keepdims=True)
    pos1 = jnp.sum((po + s) * c1, axis=1, keepdims=True)
    pos_ref[...] = jnp.concatenate([pos0, pos1], axis=1).astype(_i32)

    po_t = jnp.reshape(po / float(BM), (E, 1))
    jf = lax.broadcasted_iota(_i32, (E, 192), 1).astype(_f32)
    cnt = jnp.sum((po_t <= jf).astype(_f32), axis=0, keepdims=True)
    etile_ref[...] = jnp.minimum(cnt - 1.0, float(E - 1)).astype(_i32)


def _route(attn, res, ln2_w, Wo, Wr, interpret=False):
    return pl.pallas_call(
        _route_body,
        out_shape=(
            jax.ShapeDtypeStruct((T, H), _f32),
            jax.ShapeDtypeStruct((T, H), _f32),
            jax.ShapeDtypeStruct((T, 2), _f32),
            jax.ShapeDtypeStruct((T, 2), _i32),
            jax.ShapeDtypeStruct((1, 192), _i32),
        ),
        interpret=interpret,
    )(attn, res, ln2_w.reshape(1, H), Wo, Wr)


def _ffn_body(et_ref, xs_ref, wg_ref, wu_ref, wd_ref, o_ref):
    xs = xs_ref[...]
    g = jnp.dot(xs, wg_ref[0], preferred_element_type=_f32)
    u = jnp.dot(xs, wu_ref[0], preferred_element_type=_f32)
    o_ref[...] = jnp.dot(_silu(g) * u, wd_ref[0], preferred_element_type=_f32)


def _grouped_ffn(etile, xs, Wg, Wu, Wd, interpret=False):
    grid_spec = pltpu.PrefetchScalarGridSpec(
        num_scalar_prefetch=1,
        grid=(NT,),
        in_specs=[
            pl.BlockSpec((BM, H), lambda j, et: (j, 0)),
            pl.BlockSpec((1, H, F), lambda j, et: (et[0, j], 0, 0)),
            pl.BlockSpec((1, H, F), lambda j, et: (et[0, j], 0, 0)),
            pl.BlockSpec((1, F, H), lambda j, et: (et[0, j], 0, 0)),
        ],
        out_specs=pl.BlockSpec((BM, H), lambda j, et: (j, 0)),
    )
    return pl.pallas_call(
        _ffn_body,
        grid_spec=grid_spec,
        out_shape=jax.ShapeDtypeStruct((CAP, H), _f32),
        interpret=interpret,
    )(etile, xs, Wg, Wu, Wd)


def _sc_dispatch(x, srcidx, pos_flat):
    mesh = plsc.VectorSubcoreMesh(core_axis_name="c", subcore_axis_name="s")
    CH = 64

    @functools.partial(
        pl.kernel,
        mesh=mesh,
        out_type=jax.ShapeDtypeStruct((CAP, H), _f32),
        scratch_types=[
            pltpu.VMEM((CH,), _i32),
            pltpu.VMEM((CH,), _i32),
            pltpu.VMEM((CH, H), _f32),
            pltpu.SemaphoreType.DMA,
            pltpu.SemaphoreType.DMA,
        ],
    )
    def k(x_ref, si_ref, pos_ref, xs_ref, idx_v, pos_v, rows_v, sem1, sem2):
        wid = lax.axis_index("s") * 2 + lax.axis_index("c")
        base = wid * 128
        for c in range(2):
            off = base + c * CH
            pltpu.sync_copy(si_ref.at[pl.ds(off, CH)], idx_v)
            pltpu.async_copy(x_ref.at[idx_v], rows_v, sem1).wait()
            pltpu.sync_copy(pos_ref.at[pl.ds(off, CH)], pos_v)
            pltpu.async_copy(rows_v, xs_ref.at[pos_v], sem2).wait()

    return k(x, srcidx, pos_flat)


def _sc_combine(o_sorted, pos0, pos1):
    mesh = plsc.VectorSubcoreMesh(core_axis_name="c", subcore_axis_name="s")
    CH = 64

    @functools.partial(
        pl.kernel,
        mesh=mesh,
        out_type=(jax.ShapeDtypeStruct((T, H), _f32),
                  jax.ShapeDtypeStruct((T, H), _f32)),
        scratch_types=[
            pltpu.VMEM((CH,), _i32),
            pltpu.VMEM((CH, H), _f32),
            pltpu.SemaphoreType.DMA,
        ],
    )
    def k(o_ref, p0_ref, p1_ref, g0_ref, g1_ref, idx_v, rows_v, sem):
        wid = lax.axis_index("s") * 2 + lax.axis_index("c")
        base = wid * CH
        pltpu.sync_copy(p0_ref.at[pl.ds(base, CH)], idx_v)
        pltpu.async_copy(o_ref.at[idx_v], rows_v, sem).wait()
        pltpu.sync_copy(rows_v, g0_ref.at[pl.ds(base, CH)])
        pltpu.sync_copy(p1_ref.at[pl.ds(base, CH)], idx_v)
        pltpu.async_copy(o_ref.at[idx_v], rows_v, sem).wait()
        pltpu.sync_copy(rows_v, g1_ref.at[pl.ds(base, CH)])

    return k(o_sorted, pos0, pos1)


def _final_body(h_ref, xn_ref, wsg_ref, wsu_ref, wsd_ref,
                g0_ref, g1_ref, tw_ref, out_ref):
    xn = xn_ref[...]
    a = jnp.dot(xn, wsg_ref[...], preferred_element_type=_f32)
    b = jnp.dot(xn, wsu_ref[...], preferred_element_type=_f32)
    sh = jnp.dot(_silu(a) * b, wsd_ref[...], preferred_element_type=_f32)
    w0 = tw_ref[:, 0:1]
    w1 = tw_ref[:, 1:2]
    out_ref[...] = h_ref[...] + sh + w0 * g0_ref[...] + w1 * g1_ref[...]


def _final(h, xn, Wsg, Wsu, Wsd, g0, g1, topw, interpret=False):
    RB = 512
    nb = T // RB
    return pl.pallas_call(
        _final_body,
        grid=(nb,),
        in_specs=[
            pl.BlockSpec((RB, H), lambda i: (i, 0)),
            pl.BlockSpec((RB, H), lambda i: (i, 0)),
            pl.BlockSpec((H, FS), lambda i: (0, 0)),
            pl.BlockSpec((H, FS), lambda i: (0, 0)),
            pl.BlockSpec((FS, H), lambda i: (0, 0)),
            pl.BlockSpec((RB, H), lambda i: (i, 0)),
            pl.BlockSpec((RB, H), lambda i: (i, 0)),
            pl.BlockSpec((RB, 2), lambda i: (i, 0)),
        ],
        out_specs=pl.BlockSpec((RB, H), lambda i: (i, 0)),
        out_shape=jax.ShapeDtypeStruct((T, H), _f32),
        interpret=interpret,
    )(h, xn, Wsg, Wsu, Wsd, g0, g1, topw)


def kernel(hidden_states, attention_mask, position_ids, ln1_w, ln2_w,
           Wq, Wk, Wv, Wo, Wr, Wg, Wu, Wd, Wsg, Wsu, Wsd):
    x = hidden_states.reshape(T, H)

    pid = position_ids.reshape(T).astype(_f32)
    inv = 1.0 / (ROPE_BASE ** (jnp.arange(0, HD, 2, dtype=_f32) / HD))
    freqs = pid[:, None] * inv
    emb = jnp.concatenate([freqs, freqs], axis=1)
    cos, sin = jnp.cos(emb), jnp.sin(emb)

    attn = _attention(x, cos, sin, ln1_w, Wq, Wk, Wv)
    h, xn, topw, pos2, etile = _route(attn, x, ln2_w, Wo, Wr)

    srcidx = (jnp.arange(2 * T, dtype=_i32) // 2)
    xs = _sc_dispatch(xn, srcidx, pos2.reshape(-1))
    o_sorted = _grouped_ffn(etile, xs, Wg, Wu, Wd)
    g0, g1 = _sc_combine(o_sorted, pos2[:, 0], pos2[:, 1])

    out = _final(h, xn, Wsg, Wsu, Wsd, g0, g1, topw)
    return out.reshape(1, T, H)
